# baseline (speedup 1.0000x reference)
import functools

import jax
import jax.numpy as jnp
from jax import lax
from jax.experimental import pallas as pl
from jax.experimental.pallas import tpu as pltpu
from jax.experimental.pallas import tpu_sc as plsc

NC = 2
NS = 16
NW = NC * NS

VOCAB = 100000
D = 64
B = 4096
H = 50
NITEM = 2 * H
NIB = 5
NOB = 2


def _make_gather():
  mesh = plsc.VectorSubcoreMesh(
      core_axis_name="c", subcore_axis_name="s",
      num_cores=NC, num_subcores=NS)

  @functools.partial(
      pl.kernel,
      mesh=mesh,
      out_type=jax.ShapeDtypeStruct((H, D, B), jnp.float32),
      scratch_types=[
          pltpu.VMEM((VOCAB,), jnp.float32),
          pltpu.VMEM((B,), jnp.int32),
          pltpu.VMEM((B,), jnp.int32),
          pltpu.VMEM((B,), jnp.int32),
          pltpu.VMEM((B,), jnp.int32),
          pltpu.VMEM((B,), jnp.int32),
          pltpu.VMEM((B,), jnp.float32),
          pltpu.VMEM((B,), jnp.float32),
          pltpu.SemaphoreType.DMA,
          pltpu.SemaphoreType.DMA,
          pltpu.SemaphoreType.DMA,
          pltpu.SemaphoreType.DMA,
          pltpu.SemaphoreType.DMA,
          pltpu.SemaphoreType.DMA,
          pltpu.SemaphoreType.DMA,
      ],
      compiler_params=pltpu.CompilerParams(
          use_tc_tiling_on_sc=True, needs_layout_passes=False),
  )
  def gather_kernel(table_hbm, idx_hbm, out_hbm, row_v,
                    idxrow0, idxrow1, idxrow2, idxrow3, idxrow4,
                    outrow0, outrow1,
                    isem0, isem1, isem2, isem3, isem4, osem0, osem1):
    wid = lax.axis_index("s") * NC + lax.axis_index("c")
    idxrows = (idxrow0, idxrow1, idxrow2, idxrow3, idxrow4)
    outrows = (outrow0, outrow1)
    isems = (isem0, isem1, isem2, isem3, isem4)
    osems = (osem0, osem1)

    def idx_row_start(item, buf):
      h = item - H * (item >= H).astype(jnp.int32)
      pltpu.async_copy(
          idx_hbm.at[pl.ds(h * B, B)], idxrows[buf], isems[buf])

    idx_row_start(jnp.int32(0), 0)
    idx_row_start(jnp.int32(1), 1)
    idx_row_start(jnp.int32(2), 2)

    @pl.loop(0, NITEM, step=NIB * NOB)
    def block(i0):
      for k in range(NIB * NOB):
        ib = k % NIB
        ob = k % NOB
        i = i0 + k
        second = (i >= H).astype(jnp.int32)
        d = wid + NW * second

        @pl.when(jnp.logical_or(i == 0, i == H))
        def _():
          pltpu.sync_copy(table_hbm.at[d, :], row_v)

        @pl.when(i + 3 < NITEM)
        def _():
          idx_row_start(i + 3, (k + 3) % NIB)

        pltpu.make_async_copy(
            idx_hbm.at[pl.ds(0, B)], idxrows[ib], isems[ib]).wait()

        @pl.when(i >= NOB)
        def _():
          pltpu.make_async_copy(
              outrows[ob], out_hbm.at[0, 0, :], osems[ob]).wait()

        U = 16

        @pl.loop(0, B // (16 * U))
        def gather(gg):
          base = gg * (16 * U)
          ivs = [idxrows[ib][pl.ds(base + u * 16, 16)] for u in range(U)]
          vals = [plsc.load_gather(row_v, [iv]) for iv in ivs]
          for u in range(U):
            outrows[ob][pl.ds(base + u * 16, 16)] = vals[u]

        h = i - H * second
        pltpu.async_copy(outrows[ob], out_hbm.at[h, d, :], osems[ob])

    pltpu.make_async_copy(outrows[0], out_hbm.at[0, 0, :], osems[0]).wait()
    pltpu.make_async_copy(outrows[1], out_hbm.at[0, 0, :], osems[1]).wait()

  return gather_kernel


_gather = _make_gather()


@jax.jit
def kernel(shifts, idx):
  table_t = shifts.T
  idx_flat = idx.T.reshape(H * B)
  p = _gather(table_t, idx_flat)
  return jnp.transpose(p, (2, 0, 1))


# speedup vs baseline: 1.0088x; 1.0088x over previous
import functools

import jax
import jax.numpy as jnp
from jax import lax
from jax.experimental import pallas as pl
from jax.experimental.pallas import tpu as pltpu
from jax.experimental.pallas import tpu_sc as plsc

NC = 2
NS = 16
NW = NC * NS

VOCAB = 100000
D = 64
B = 4096
H = 50
NITEM = 2 * H
NIB = 5
NOB = 2


def _make_gather():
  mesh = plsc.VectorSubcoreMesh(
      core_axis_name="c", subcore_axis_name="s",
      num_cores=NC, num_subcores=NS)

  @functools.partial(
      pl.kernel,
      mesh=mesh,
      out_type=jax.ShapeDtypeStruct((H, D, B), jnp.float32),
      scratch_types=[
          pltpu.VMEM((VOCAB,), jnp.float32),
          pltpu.VMEM((B,), jnp.int32),
          pltpu.VMEM((B,), jnp.int32),
          pltpu.VMEM((B,), jnp.int32),
          pltpu.VMEM((B,), jnp.int32),
          pltpu.VMEM((B,), jnp.int32),
          pltpu.VMEM((B,), jnp.float32),
          pltpu.VMEM((B,), jnp.float32),
          pltpu.SemaphoreType.DMA,
          pltpu.SemaphoreType.DMA,
          pltpu.SemaphoreType.DMA,
          pltpu.SemaphoreType.DMA,
          pltpu.SemaphoreType.DMA,
          pltpu.SemaphoreType.DMA,
          pltpu.SemaphoreType.DMA,
      ],
      compiler_params=pltpu.CompilerParams(
          use_tc_tiling_on_sc=True, needs_layout_passes=False),
  )
  def gather_kernel(table_hbm, idx_hbm, out_hbm, row_v,
                    idxrow0, idxrow1, idxrow2, idxrow3, idxrow4,
                    outrow0, outrow1,
                    isem0, isem1, isem2, isem3, isem4, osem0, osem1):
    wid = lax.axis_index("s") * NC + lax.axis_index("c")
    idxrows = (idxrow0, idxrow1, idxrow2, idxrow3, idxrow4)
    outrows = (outrow0, outrow1)
    isems = (isem0, isem1, isem2, isem3, isem4)
    osems = (osem0, osem1)

    def idx_row_start(item, buf):
      h = item - H * (item >= H).astype(jnp.int32)
      pltpu.async_copy(idx_hbm.at[h, :], idxrows[buf], isems[buf])

    idx_row_start(jnp.int32(0), 0)
    idx_row_start(jnp.int32(1), 1)
    idx_row_start(jnp.int32(2), 2)

    @pl.loop(0, NITEM, step=NIB * NOB)
    def block(i0):
      for k in range(NIB * NOB):
        ib = k % NIB
        ob = k % NOB
        i = i0 + k
        second = (i >= H).astype(jnp.int32)
        d = wid + NW * second

        @pl.when(jnp.logical_or(i == 0, i == H))
        def _():
          pltpu.sync_copy(table_hbm.at[d, :], row_v)

        @pl.when(i + 3 < NITEM)
        def _():
          idx_row_start(i + 3, (k + 3) % NIB)

        pltpu.make_async_copy(
            idx_hbm.at[0, :], idxrows[ib], isems[ib]).wait()

        @pl.when(i >= NOB)
        def _():
          pltpu.make_async_copy(
              outrows[ob], out_hbm.at[0, 0, :], osems[ob]).wait()

        U = 8

        @pl.loop(0, B // (16 * U))
        def gather(gg):
          base = gg * (16 * U)
          ivs = [idxrows[ib][pl.ds(base + u * 16, 16)] for u in range(U)]
          vals = [plsc.load_gather(row_v, [iv]) for iv in ivs]
          for u in range(U):
            outrows[ob][pl.ds(base + u * 16, 16)] = vals[u]

        h = i - H * second
        pltpu.async_copy(outrows[ob], out_hbm.at[h, d, :], osems[ob])

    pltpu.make_async_copy(outrows[0], out_hbm.at[0, 0, :], osems[0]).wait()
    pltpu.make_async_copy(outrows[1], out_hbm.at[0, 0, :], osems[1]).wait()

  return gather_kernel


_gather = _make_gather()


@jax.jit
def kernel(shifts, idx):
  table_t = shifts.T
  idx_t = idx.T
  p = _gather(table_t, idx_t)
  return jnp.transpose(p, (2, 0, 1))


# speedup vs baseline: 1.2352x; 1.2244x over previous
import functools

import jax
import jax.numpy as jnp
from jax import lax
from jax.experimental import pallas as pl
from jax.experimental.pallas import tpu as pltpu
from jax.experimental.pallas import tpu_sc as plsc

NC = 2
NS = 16
NW = NC * NS

VOCAB = 100000
D = 64
B = 4096
H = 50
NITEM = 2 * H
NIB = 5
NOB = 2


def _make_gather():
  mesh = plsc.VectorSubcoreMesh(
      core_axis_name="c", subcore_axis_name="s",
      num_cores=NC, num_subcores=NS)

  @functools.partial(
      pl.kernel,
      mesh=mesh,
      out_type=jax.ShapeDtypeStruct((H, D, B), jnp.float32),
      scratch_types=[
          pltpu.VMEM((VOCAB,), jnp.float32),
          pltpu.VMEM((B,), jnp.int32),
          pltpu.VMEM((B,), jnp.int32),
          pltpu.VMEM((B,), jnp.int32),
          pltpu.VMEM((B,), jnp.int32),
          pltpu.VMEM((B,), jnp.int32),
          pltpu.VMEM((B,), jnp.float32),
          pltpu.VMEM((B,), jnp.float32),
          pltpu.SemaphoreType.DMA,
          pltpu.SemaphoreType.DMA,
          pltpu.SemaphoreType.DMA,
          pltpu.SemaphoreType.DMA,
          pltpu.SemaphoreType.DMA,
          pltpu.SemaphoreType.DMA,
          pltpu.SemaphoreType.DMA,
      ],
      compiler_params=pltpu.CompilerParams(
          use_tc_tiling_on_sc=True, needs_layout_passes=False),
  )
  def gather_kernel(table_hbm, idx_hbm, out_hbm, row_v,
                    idxrow0, idxrow1, idxrow2, idxrow3, idxrow4,
                    outrow0, outrow1,
                    isem0, isem1, isem2, isem3, isem4, osem0, osem1):
    wid = lax.axis_index("s") * NC + lax.axis_index("c")
    idxrows = (idxrow0, idxrow1, idxrow2, idxrow3, idxrow4)
    outrows = (outrow0, outrow1)
    isems = (isem0, isem1, isem2, isem3, isem4)
    osems = (osem0, osem1)

    off = (wid * 3) // 2

    def actual_of(j):
      a = j + off
      return a - NITEM * (a >= NITEM).astype(jnp.int32)

    def idx_row_start(j, buf):
      a = actual_of(j)
      h = a - H * (a >= H).astype(jnp.int32)
      pltpu.async_copy(idx_hbm.at[h, :], idxrows[buf], isems[buf])

    idx_row_start(jnp.int32(0), 0)
    idx_row_start(jnp.int32(1), 1)
    idx_row_start(jnp.int32(2), 2)

    @pl.loop(0, NITEM, step=NIB * NOB)
    def block(i0):
      for k in range(NIB * NOB):
        ib = k % NIB
        ob = k % NOB
        i = i0 + k
        a = actual_of(i)
        second = (a >= H).astype(jnp.int32)
        d = wid + NW * second

        @pl.when(jnp.logical_or(i == 0, a - H * second == 0))
        def _():
          pltpu.sync_copy(table_hbm.at[d, :], row_v)

        @pl.when(i + 3 < NITEM)
        def _():
          idx_row_start(i + 3, (k + 3) % NIB)

        pltpu.make_async_copy(
            idx_hbm.at[0, :], idxrows[ib], isems[ib]).wait()

        @pl.when(i >= NOB)
        def _():
          pltpu.make_async_copy(
              outrows[ob], out_hbm.at[0, 0, :], osems[ob]).wait()

        U = 8

        @pl.loop(0, B // (16 * U))
        def gather(gg):
          base = gg * (16 * U)
          ivs = [idxrows[ib][pl.ds(base + u * 16, 16)] for u in range(U)]
          vals = [plsc.load_gather(row_v, [iv]) for iv in ivs]
          for u in range(U):
            outrows[ob][pl.ds(base + u * 16, 16)] = vals[u]

        h = a - H * second
        pltpu.async_copy(outrows[ob], out_hbm.at[h, d, :], osems[ob])

    pltpu.make_async_copy(outrows[0], out_hbm.at[0, 0, :], osems[0]).wait()
    pltpu.make_async_copy(outrows[1], out_hbm.at[0, 0, :], osems[1]).wait()

  return gather_kernel


_gather = _make_gather()


@jax.jit
def kernel(shifts, idx):
  table_t = shifts.T
  idx_t = idx.T
  p = _gather(table_t, idx_t)
  return jnp.transpose(p, (2, 0, 1))


# speedup vs baseline: 1.2449x; 1.0079x over previous
import functools

import jax
import jax.numpy as jnp
from jax import lax
from jax.experimental import pallas as pl
from jax.experimental.pallas import tpu as pltpu
from jax.experimental.pallas import tpu_sc as plsc

NC = 2
NS = 16
NW = NC * NS

VOCAB = 100000
D = 64
B = 4096
H = 50
NITEM = 2 * H
NIB = 5
NOB = 2


def _make_gather():
  mesh = plsc.VectorSubcoreMesh(
      core_axis_name="c", subcore_axis_name="s",
      num_cores=NC, num_subcores=NS)

  @functools.partial(
      pl.kernel,
      mesh=mesh,
      out_type=jax.ShapeDtypeStruct((H, D, B), jnp.float32),
      scratch_types=[
          pltpu.VMEM((VOCAB,), jnp.float32),
          pltpu.VMEM((B,), jnp.int32),
          pltpu.VMEM((B,), jnp.int32),
          pltpu.VMEM((B,), jnp.int32),
          pltpu.VMEM((B,), jnp.int32),
          pltpu.VMEM((B,), jnp.int32),
          pltpu.VMEM((B,), jnp.float32),
          pltpu.VMEM((B,), jnp.float32),
          pltpu.SemaphoreType.DMA,
          pltpu.SemaphoreType.DMA,
          pltpu.SemaphoreType.DMA,
          pltpu.SemaphoreType.DMA,
          pltpu.SemaphoreType.DMA,
          pltpu.SemaphoreType.DMA,
          pltpu.SemaphoreType.DMA,
      ],
      compiler_params=pltpu.CompilerParams(
          use_tc_tiling_on_sc=True, needs_layout_passes=False),
  )
  def gather_kernel(table_hbm, idx_hbm, out_hbm, row_v,
                    idxrow0, idxrow1, idxrow2, idxrow3, idxrow4,
                    outrow0, outrow1,
                    isem0, isem1, isem2, isem3, isem4, osem0, osem1):
    wid = lax.axis_index("s") * NC + lax.axis_index("c")
    idxrows = (idxrow0, idxrow1, idxrow2, idxrow3, idxrow4)
    outrows = (outrow0, outrow1)
    isems = (isem0, isem1, isem2, isem3, isem4)
    osems = (osem0, osem1)

    off = (wid * 3) // 2

    def actual_of(j):
      a = j + off
      return a - NITEM * (a >= NITEM).astype(jnp.int32)

    def idx_row_start(j, buf):
      a = actual_of(j)
      h = a - H * (a >= H).astype(jnp.int32)
      pltpu.async_copy(idx_hbm.at[h, :], idxrows[buf], isems[buf])

    idx_row_start(jnp.int32(0), 0)
    idx_row_start(jnp.int32(1), 1)
    idx_row_start(jnp.int32(2), 2)
    idx_row_start(jnp.int32(3), 3)

    @pl.loop(0, NITEM, step=NIB * NOB)
    def block(i0):
      for k in range(NIB * NOB):
        ib = k % NIB
        ob = k % NOB
        i = i0 + k
        a = actual_of(i)
        second = (a >= H).astype(jnp.int32)
        d = wid + NW * second

        @pl.when(jnp.logical_or(i == 0, a - H * second == 0))
        def _():
          pltpu.sync_copy(table_hbm.at[d, :], row_v)

        @pl.when(i + 4 < NITEM)
        def _():
          idx_row_start(i + 4, (k + 4) % NIB)

        pltpu.make_async_copy(
            idx_hbm.at[0, :], idxrows[ib], isems[ib]).wait()

        @pl.when(i >= NOB)
        def _():
          pltpu.make_async_copy(
              outrows[ob], out_hbm.at[0, 0, :], osems[ob]).wait()

        U = 8

        @pl.loop(0, B // (16 * U))
        def gather(gg):
          base = gg * (16 * U)
          ivs = [idxrows[ib][pl.ds(base + u * 16, 16)] for u in range(U)]
          vals = [plsc.load_gather(row_v, [iv]) for iv in ivs]
          for u in range(U):
            outrows[ob][pl.ds(base + u * 16, 16)] = vals[u]

        h = a - H * second
        pltpu.async_copy(outrows[ob], out_hbm.at[h, d, :], osems[ob])

    pltpu.make_async_copy(outrows[0], out_hbm.at[0, 0, :], osems[0]).wait()
    pltpu.make_async_copy(outrows[1], out_hbm.at[0, 0, :], osems[1]).wait()

  return gather_kernel


_gather = _make_gather()


@jax.jit
def kernel(shifts, idx):
  table_t = shifts.T
  idx_t = idx.T
  p = _gather(table_t, idx_t)
  return jnp.transpose(p, (2, 0, 1))
